# baseline (speedup 1.0000x reference)
import jax
import jax.numpy as jnp
from jax import lax
from jax.experimental import pallas as pl
from jax.experimental.pallas import tpu as pltpu
from jax.experimental.pallas import tpu_sc as plsc

R = 32
N = 1_000_000
CH = 50_000
NCH = N // CH
BLK = 2_000
NBLK = CH // BLK
LANES = 16
NACC = 25


def _tree_sum(vs):
    vs = list(vs)
    while len(vs) > 1:
        nxt = [a + b for a, b in zip(vs[::2], vs[1::2])]
        if len(vs) % 2:
            nxt.append(vs[-1])
        vs = nxt
    return vs[0]


def _region_sum(buf, base, nvregs):
    iters = nvregs // NACC
    def body(i, accs):
        off = base + i * (NACC * LANES)
        return tuple(
            accs[j] + buf[pl.ds(off + j * LANES, LANES)] for j in range(NACC)
        )
    init = tuple(jnp.zeros((LANES,), jnp.float32) for _ in range(NACC))
    accs = lax.fori_loop(0, iters, body, init)
    return jnp.sum(_tree_sum(accs))


def _fine_count(buf, start, prefix, tv):
    def body(i, carry):
        run, c1, c2 = carry
        v = buf[pl.ds(start + i * LANES, LANES)]
        absc = plsc.cumsum(v) + jnp.full((LANES,), run)
        m = absc < tv
        c1 = c1 + m.astype(jnp.int32)
        c2 = c2 + m.astype(jnp.int32)
        return run + jnp.sum(v), c1, c2
    z = jnp.zeros((LANES,), jnp.int32)
    _, c1, c2 = lax.fori_loop(0, BLK // LANES, body, (prefix, z, z))
    return jnp.sum(c1), jnp.sum(c2)


def _scalar_scan(sums, t):
    run = jnp.float32(0.0)
    nbelow = jnp.int32(0)
    pfx = jnp.float32(0.0)
    for s in sums:
        run = run + s
        below = run < t
        nbelow = nbelow + below.astype(jnp.int32)
        pfx = pfx + jnp.where(below, s, jnp.float32(0.0))
    return nbelow, pfx


def _sc_body(x_hbm, u_hbm, out_hbm, bufa, bufb, u_v, flag_v, sema, semb):
    wid = lax.axis_index("s") * 2 + lax.axis_index("c")
    row = wid * N
    pltpu.sync_copy(u_hbm.at[pl.ds(wid * LANES, LANES)], u_v)

    bufs = (bufa, bufb)
    sems = (sema, semb)

    h = [None] * NCH
    h[0] = pltpu.async_copy(x_hbm.at[pl.ds(row, CH)], bufs[0], sems[0])
    chunk_sums = []
    for c in range(NCH):
        if c + 1 < NCH:
            h[c + 1] = pltpu.async_copy(
                x_hbm.at[pl.ds(row + (c + 1) * CH, CH)],
                bufs[(c + 1) % 2], sems[(c + 1) % 2])
        h[c].wait()
        chunk_sums.append(_region_sum(bufs[c % 2], 0, CH // LANES))
    total = _tree_sum(chunk_sums)

    u_s = u_v[...][0]
    t = u_s * total
    tv = jnp.full((LANES,), t)

    nfull, pfx = _scalar_scan(chunk_sums, t)
    c_star = jnp.minimum(nfull, NCH - 1)

    pltpu.sync_copy(x_hbm.at[pl.ds(row + c_star * CH, CH)], bufs[0])
    block_sums = [_region_sum(bufs[0], b * BLK, BLK // LANES)
                  for b in range(NBLK)]
    nb, bpfx = _scalar_scan(block_sums, t - pfx)
    b_star = jnp.minimum(nb, NBLK - 1)
    pfx2 = pfx + bpfx

    cnt1, cnt2 = _fine_count(bufs[0], b_star * BLK, pfx2, tv)
    idx1 = c_star * CH + b_star * BLK + cnt1
    idx2 = c_star * CH + b_star * BLK + cnt2

    neq = idx1 != idx2
    flag_v[...] = jnp.full((LANES,), jnp.where(neq, 1.0, 0.0)
                           .astype(jnp.float32))
    pltpu.sync_copy(flag_v, out_hbm.at[pl.ds(wid * LANES, LANES)])


def kernel(x):
    u = jax.random.uniform(jax.random.key(42), (R, 1), dtype=jnp.float32)
    ub = jnp.broadcast_to(u, (R, LANES)).reshape(R * LANES)
    mesh = plsc.VectorSubcoreMesh(core_axis_name="c", subcore_axis_name="s",
                                  num_cores=2, num_subcores=16)
    run = pl.kernel(
        _sc_body,
        out_type=jax.ShapeDtypeStruct((R * LANES,), jnp.float32),
        mesh=mesh,
        scratch_types=[
            pltpu.VMEM((CH,), jnp.float32),
            pltpu.VMEM((CH,), jnp.float32),
            pltpu.VMEM((LANES,), jnp.float32),
            pltpu.VMEM((LANES,), jnp.float32),
            pltpu.SemaphoreType.DMA,
            pltpu.SemaphoreType.DMA,
        ],
        compiler_params=pltpu.CompilerParams(needs_layout_passes=False),
    )
    flags = run(x.reshape(R * N), ub)
    return jnp.any(flags != 0.0).astype(jnp.float32)
